# Initial kernel scaffold; baseline (speedup 1.0000x reference)
import functools

import jax
import jax.numpy as jnp
from jax import lax
from jax.experimental import pallas as pl
from jax.experimental.pallas import tpu as pltpu
from jax.experimental.pallas import tpu_sc as plsc

N = 10000
E = 320000
F_IN = 16
F_OUT = 16
EPS = 1e-5

TE = 2000
GRID = E // TE

NC = 2
NS = 16
NW = NC * NS
PER_W = E // NW
SCCH = 2000
NCHUNK = PER_W // SCCH

_ARB = pltpu.CompilerParams(dimension_semantics=("arbitrary",))


def _z0(ef, w0t_ref, b0_ref):
    return (ef[:, 0:1] * w0t_ref[0:1, :]
            + ef[:, 1:2] * w0t_ref[1:2, :]
            + ef[:, 2:3] * w0t_ref[2:3, :]
            + b0_ref[...])


def _acc_stats(i, z, s_ref, q_ref):
    ps = jnp.sum(z, axis=0, keepdims=True)
    pq = jnp.sum(z * z, axis=0, keepdims=True)

    @pl.when(i == 0)
    def _():
        s_ref[...] = ps
        q_ref[...] = pq

    @pl.when(i > 0)
    def _():
        s_ref[...] = s_ref[...] + ps
        q_ref[...] = q_ref[...] + pq


def _bn_relu(z, s_ref, q_ref, g_ref, be_ref):
    m = s_ref[...] * (1.0 / E)
    v = q_ref[...] * (1.0 / E) - m * m
    a = g_ref[...] * lax.rsqrt(v + EPS)
    c = be_ref[...] - m * a
    return jnp.maximum(z * a + c, 0.0)


def _stats0_body(ef_ref, w0t_ref, b0_ref, s_ref, q_ref):
    i = pl.program_id(0)
    z = _z0(ef_ref[...], w0t_ref, b0_ref)
    _acc_stats(i, z, s_ref, q_ref)


def _layer1_body(ef_ref, w0t_ref, b0_ref, s0_ref, q0_ref, g0_ref, be0_ref,
                 w1t_ref, b1_ref, z1_ref, s1_ref, q1_ref):
    i = pl.program_id(0)
    z0 = _z0(ef_ref[...], w0t_ref, b0_ref)
    h0 = _bn_relu(z0, s0_ref, q0_ref, g0_ref, be0_ref)
    z1 = jnp.dot(h0, w1t_ref[...], preferred_element_type=jnp.float32) + b1_ref[...]
    z1_ref[...] = z1
    _acc_stats(i, z1, s1_ref, q1_ref)


def _mid_body(z_ref, s_ref, q_ref, g_ref, be_ref, wt_ref, b_ref,
              zo_ref, so_ref, qo_ref):
    i = pl.program_id(0)
    h = _bn_relu(z_ref[...], s_ref, q_ref, g_ref, be_ref)
    z = jnp.dot(h, wt_ref[...], preferred_element_type=jnp.float32) + b_ref[...]
    zo_ref[...] = z
    _acc_stats(i, z, so_ref, qo_ref)


def _final_body(z3_ref, s_ref, q_ref, g_ref, be_ref, wft_ref, bf_ref,
                x_ref, msg_ref):
    h3 = _bn_relu(z3_ref[...], s_ref, q_ref, g_ref, be_ref)
    wv = jax.nn.sigmoid(
        jnp.dot(h3, wft_ref[...], preferred_element_type=jnp.float32)
        + bf_ref[...])
    x = x_ref[...]
    acc = x[:, 0:1] * wv[:, 0:F_OUT]
    for k in range(1, F_IN):
        acc = acc + x[:, k:k + 1] * wv[:, k * F_OUT:(k + 1) * F_OUT]
    msg_ref[...] = acc


def _epi_body(p_ref, d_ref, bias_ref, out_ref):
    p = p_ref[...]
    d = d_ref[...]
    deg = jnp.maximum(d[0] + d[1], 1.0)
    out_ref[...] = (p[0] + p[1]) / deg + bias_ref[...]


def _row(x):
    return x.reshape(1, -1)


def _const_spec(shape):
    return pl.BlockSpec(shape, lambda i: (0,) * len(shape))


def _stats_out(ch):
    return ([jax.ShapeDtypeStruct((1, ch), jnp.float32)] * 2,
            [_const_spec((1, ch))] * 2)


def _stats0(efeat, w0t, b0r):
    out_shape, out_specs = _stats_out(256)
    return pl.pallas_call(
        _stats0_body,
        grid=(GRID,),
        in_specs=[pl.BlockSpec((TE, 3), lambda i: (i, 0)),
                  _const_spec((3, 256)), _const_spec((1, 256))],
        out_specs=out_specs,
        out_shape=out_shape,
        compiler_params=_ARB,
    )(efeat, w0t, b0r)


def _layer1(efeat, w0t, b0r, s0, q0, g0r, be0r, w1t, b1r):
    out_shape = [jax.ShapeDtypeStruct((E, 128), jnp.float32)]
    out_specs = [pl.BlockSpec((TE, 128), lambda i: (i, 0))]
    so, sspec = _stats_out(128)
    return pl.pallas_call(
        _layer1_body,
        grid=(GRID,),
        in_specs=[pl.BlockSpec((TE, 3), lambda i: (i, 0)),
                  _const_spec((3, 256)), _const_spec((1, 256)),
                  _const_spec((1, 256)), _const_spec((1, 256)),
                  _const_spec((1, 256)), _const_spec((1, 256)),
                  _const_spec((256, 128)), _const_spec((1, 128))],
        out_specs=out_specs + sspec,
        out_shape=out_shape + so,
        compiler_params=_ARB,
    )(efeat, w0t, b0r, s0, q0, g0r, be0r, w1t, b1r)


def _mid(z, s, q, gr, ber, wt, br, cin, cout):
    out_shape = [jax.ShapeDtypeStruct((E, cout), jnp.float32)]
    out_specs = [pl.BlockSpec((TE, cout), lambda i: (i, 0))]
    so, sspec = _stats_out(cout)
    return pl.pallas_call(
        _mid_body,
        grid=(GRID,),
        in_specs=[pl.BlockSpec((TE, cin), lambda i: (i, 0)),
                  _const_spec((1, cin)), _const_spec((1, cin)),
                  _const_spec((1, cin)), _const_spec((1, cin)),
                  _const_spec((cin, cout)), _const_spec((1, cout))],
        out_specs=out_specs + sspec,
        out_shape=out_shape + so,
        compiler_params=_ARB,
    )(z, s, q, gr, ber, wt, br)


def _final(z3, s3, q3, g3r, be3r, wft, bfr, x_src):
    return pl.pallas_call(
        _final_body,
        grid=(GRID,),
        in_specs=[pl.BlockSpec((TE, 32), lambda i: (i, 0)),
                  _const_spec((1, 32)), _const_spec((1, 32)),
                  _const_spec((1, 32)), _const_spec((1, 32)),
                  _const_spec((32, 256)), _const_spec((1, 256)),
                  pl.BlockSpec((TE, F_IN), lambda i: (i, 0))],
        out_specs=pl.BlockSpec((TE, F_OUT), lambda i: (i, 0)),
        out_shape=jax.ShapeDtypeStruct((E, F_OUT), jnp.float32),
        compiler_params=_ARB,
    )(z3, s3, q3, g3r, be3r, wft, bfr, x_src)


def _epilogue(p2, d2, bias8):
    return pl.pallas_call(
        _epi_body,
        out_shape=jax.ShapeDtypeStruct((N // 8, 128), jnp.float32),
    )(p2, d2, bias8)


_SC_MESH = plsc.VectorSubcoreMesh(core_axis_name="c", subcore_axis_name="s")


@functools.partial(
    pl.kernel,
    mesh=_SC_MESH,
    out_type=jax.ShapeDtypeStruct((E, F_IN), jnp.float32),
    scratch_types=[pltpu.VMEM((SCCH,), jnp.int32),
                   pltpu.VMEM((SCCH, F_IN), jnp.float32),
                   pltpu.SemaphoreType.DMA],
)
def _sc_gather(nfeat_hbm, src_hbm, out_hbm, idx_v, rows_v, sem):
    wid = lax.axis_index("s") * NC + lax.axis_index("c")
    base = wid * PER_W

    def body(j, carry):
        off = base + j * SCCH
        pltpu.sync_copy(src_hbm.at[pl.ds(off, SCCH)], idx_v)
        pltpu.async_copy(nfeat_hbm.at[idx_v], rows_v, sem).wait()
        pltpu.sync_copy(rows_v, out_hbm.at[pl.ds(off, SCCH)])
        return carry

    lax.fori_loop(0, NCHUNK, body, 0)


@functools.partial(
    pl.kernel,
    mesh=_SC_MESH,
    out_type=[jax.ShapeDtypeStruct((NC, N, F_OUT), jnp.float32),
              jax.ShapeDtypeStruct((NC, N, F_OUT), jnp.float32)],
    scratch_types=[pltpu.VMEM((SCCH,), jnp.int32),
                   pltpu.VMEM((SCCH, F_OUT), jnp.float32),
                   pltpu.VMEM((SCCH, F_OUT), jnp.float32),
                   pltpu.VMEM_SHARED((N, F_OUT), jnp.float32),
                   pltpu.VMEM_SHARED((N, F_OUT), jnp.float32)],
)
def _sc_scatter(msg_hbm, dst_hbm, zeros_hbm, ones_hbm,
                neigh_out, deg_out, idx_v, m_v, ones_v, acc_sh, deg_sh):
    c = lax.axis_index("c")
    s = lax.axis_index("s")
    wid = s * NC + c
    base = wid * PER_W

    @pl.when(s == 0)
    def _():
        pltpu.sync_copy(zeros_hbm, acc_sh)
        pltpu.sync_copy(zeros_hbm, deg_sh)

    pltpu.sync_copy(ones_hbm, ones_v)
    plsc.subcore_barrier()

    def body(j, carry):
        off = base + j * SCCH
        pltpu.sync_copy(dst_hbm.at[pl.ds(off, SCCH)], idx_v)
        pltpu.sync_copy(msg_hbm.at[pl.ds(off, SCCH)], m_v)
        pltpu.sync_copy(m_v, acc_sh.at[idx_v], add=True)
        pltpu.sync_copy(ones_v, deg_sh.at[idx_v], add=True)
        return carry

    lax.fori_loop(0, NCHUNK, body, 0)
    plsc.subcore_barrier()

    @pl.when(s == 0)
    def _():
        pltpu.sync_copy(acc_sh, neigh_out.at[c])
        pltpu.sync_copy(deg_sh, deg_out.at[c])


def kernel(nfeat, edge_index, efeat,
           W0, b0, g0, be0,
           W1, b1, g1, be1,
           W2, b2, g2, be2,
           W3, b3, g3, be3,
           Wf, bf, bias):
    src = edge_index[0]
    dst = edge_index[1]

    w0t = W0.T
    w1t = W1.T
    w2t = W2.T
    w3t = W3.T
    wft = Wf.T

    s0, q0 = _stats0(efeat, w0t, _row(b0))
    x_src = _sc_gather(nfeat, src)

    z1, s1, q1 = _layer1(efeat, w0t, _row(b0), s0, q0, _row(g0), _row(be0),
                         w1t, _row(b1))
    z2, s2, q2 = _mid(z1, s1, q1, _row(g1), _row(be1), w2t, _row(b2), 128, 64)
    z3, s3, q3 = _mid(z2, s2, q2, _row(g2), _row(be2), w3t, _row(b3), 64, 32)
    msg = _final(z3, s3, q3, _row(g3), _row(be3), wft, _row(bf), x_src)

    zeros = jnp.zeros((N, F_OUT), jnp.float32)
    ones = jnp.ones((SCCH, F_OUT), jnp.float32)
    part, degp = _sc_scatter(msg, dst, zeros, ones)

    p2 = part.reshape(NC, N // 8, 128)
    d2 = degp.reshape(NC, N // 8, 128)
    bias8 = jnp.tile(bias, 8).reshape(1, 128)
    out = _epilogue(p2, d2, bias8)
    return out.reshape(N, F_OUT)


# speedup vs baseline: 2.3581x; 2.3581x over previous
import functools

import jax
import jax.numpy as jnp
from jax import lax
from jax.experimental import pallas as pl
from jax.experimental.pallas import tpu as pltpu
from jax.experimental.pallas import tpu_sc as plsc

N = 10000
E = 320000
F_IN = 16
F_OUT = 16
EPS = 1e-5

TE = 2000
GRID = E // TE

NC = 2
NS = 16
NW = NC * NS
PER_W = E // NW
SCCH = 2000
NCHUNK = PER_W // SCCH

_ARB = pltpu.CompilerParams(dimension_semantics=("arbitrary",))


def _z0(ef, w0t_ref, b0_ref):
    return (ef[:, 0:1] * w0t_ref[0:1, :]
            + ef[:, 1:2] * w0t_ref[1:2, :]
            + ef[:, 2:3] * w0t_ref[2:3, :]
            + b0_ref[...])


def _acc_stats(i, z, s_ref, q_ref):
    ps = jnp.sum(z, axis=0, keepdims=True)
    pq = jnp.sum(z * z, axis=0, keepdims=True)

    @pl.when(i == 0)
    def _():
        s_ref[...] = ps
        q_ref[...] = pq

    @pl.when(i > 0)
    def _():
        s_ref[...] = s_ref[...] + ps
        q_ref[...] = q_ref[...] + pq


def _bn_relu(z, s_ref, q_ref, g_ref, be_ref):
    m = s_ref[...] * (1.0 / E)
    v = q_ref[...] * (1.0 / E) - m * m
    a = g_ref[...] * lax.rsqrt(v + EPS)
    c = be_ref[...] - m * a
    return jnp.maximum(z * a + c, 0.0)


def _stats0_body(ef_ref, w0t_ref, b0_ref, s_ref, q_ref):
    i = pl.program_id(0)
    z = _z0(ef_ref[...], w0t_ref, b0_ref)
    _acc_stats(i, z, s_ref, q_ref)


def _layer1_body(ef_ref, w0t_ref, b0_ref, s0_ref, q0_ref, g0_ref, be0_ref,
                 w1t_ref, b1_ref, z1_ref, s1_ref, q1_ref):
    i = pl.program_id(0)
    z0 = _z0(ef_ref[...], w0t_ref, b0_ref)
    h0 = _bn_relu(z0, s0_ref, q0_ref, g0_ref, be0_ref)
    z1 = jnp.dot(h0, w1t_ref[...], preferred_element_type=jnp.float32) + b1_ref[...]
    z1_ref[...] = z1
    _acc_stats(i, z1, s1_ref, q1_ref)


def _mid_body(z_ref, s_ref, q_ref, g_ref, be_ref, wt_ref, b_ref,
              zo_ref, so_ref, qo_ref):
    i = pl.program_id(0)
    h = _bn_relu(z_ref[...], s_ref, q_ref, g_ref, be_ref)
    z = jnp.dot(h, wt_ref[...], preferred_element_type=jnp.float32) + b_ref[...]
    zo_ref[...] = z
    _acc_stats(i, z, so_ref, qo_ref)


def _final_body(z3_ref, s_ref, q_ref, g_ref, be_ref, wft_ref, bf_ref,
                x_ref, msg_ref):
    h3 = _bn_relu(z3_ref[...], s_ref, q_ref, g_ref, be_ref)
    wv = jax.nn.sigmoid(
        jnp.dot(h3, wft_ref[...], preferred_element_type=jnp.float32)
        + bf_ref[...])
    x = x_ref[...]
    li = lax.broadcasted_iota(jnp.int32, (F_IN, F_IN * F_OUT), 1)
    ri = lax.broadcasted_iota(jnp.int32, (F_IN, F_IN * F_OUT), 0)
    rmat = (li // F_OUT == ri).astype(jnp.float32)
    si = lax.broadcasted_iota(jnp.int32, (F_IN * F_OUT, F_OUT), 0)
    oi = lax.broadcasted_iota(jnp.int32, (F_IN * F_OUT, F_OUT), 1)
    smat = (si % F_OUT == oi).astype(jnp.float32)
    xrep = jnp.dot(x, rmat, preferred_element_type=jnp.float32)
    msg_ref[...] = jnp.dot(xrep * wv, smat,
                           preferred_element_type=jnp.float32)


def _epi_body(p_ref, d_ref, bias_ref, out_ref):
    p = p_ref[...]
    d = d_ref[...]
    deg = jnp.maximum(d[0] + d[1], 1.0)
    out_ref[...] = (p[0] + p[1]) / deg + bias_ref[...]


def _row(x):
    return x.reshape(1, -1)


def _const_spec(shape):
    return pl.BlockSpec(shape, lambda i: (0,) * len(shape))


def _stats_out(ch):
    return ([jax.ShapeDtypeStruct((1, ch), jnp.float32)] * 2,
            [_const_spec((1, ch))] * 2)


def _stats0(efeat, w0t, b0r):
    out_shape, out_specs = _stats_out(256)
    return pl.pallas_call(
        _stats0_body,
        grid=(GRID,),
        in_specs=[pl.BlockSpec((TE, 3), lambda i: (i, 0)),
                  _const_spec((3, 256)), _const_spec((1, 256))],
        out_specs=out_specs,
        out_shape=out_shape,
        compiler_params=_ARB,
    )(efeat, w0t, b0r)


def _layer1(efeat, w0t, b0r, s0, q0, g0r, be0r, w1t, b1r):
    out_shape = [jax.ShapeDtypeStruct((E, 128), jnp.float32)]
    out_specs = [pl.BlockSpec((TE, 128), lambda i: (i, 0))]
    so, sspec = _stats_out(128)
    return pl.pallas_call(
        _layer1_body,
        grid=(GRID,),
        in_specs=[pl.BlockSpec((TE, 3), lambda i: (i, 0)),
                  _const_spec((3, 256)), _const_spec((1, 256)),
                  _const_spec((1, 256)), _const_spec((1, 256)),
                  _const_spec((1, 256)), _const_spec((1, 256)),
                  _const_spec((256, 128)), _const_spec((1, 128))],
        out_specs=out_specs + sspec,
        out_shape=out_shape + so,
        compiler_params=_ARB,
    )(efeat, w0t, b0r, s0, q0, g0r, be0r, w1t, b1r)


def _mid(z, s, q, gr, ber, wt, br, cin, cout):
    out_shape = [jax.ShapeDtypeStruct((E, cout), jnp.float32)]
    out_specs = [pl.BlockSpec((TE, cout), lambda i: (i, 0))]
    so, sspec = _stats_out(cout)
    return pl.pallas_call(
        _mid_body,
        grid=(GRID,),
        in_specs=[pl.BlockSpec((TE, cin), lambda i: (i, 0)),
                  _const_spec((1, cin)), _const_spec((1, cin)),
                  _const_spec((1, cin)), _const_spec((1, cin)),
                  _const_spec((cin, cout)), _const_spec((1, cout))],
        out_specs=out_specs + sspec,
        out_shape=out_shape + so,
        compiler_params=_ARB,
    )(z, s, q, gr, ber, wt, br)


def _final(z3, s3, q3, g3r, be3r, wft, bfr, x_src):
    return pl.pallas_call(
        _final_body,
        grid=(GRID,),
        in_specs=[pl.BlockSpec((TE, 32), lambda i: (i, 0)),
                  _const_spec((1, 32)), _const_spec((1, 32)),
                  _const_spec((1, 32)), _const_spec((1, 32)),
                  _const_spec((32, 256)), _const_spec((1, 256)),
                  pl.BlockSpec((TE, F_IN), lambda i: (i, 0))],
        out_specs=pl.BlockSpec((TE, F_OUT), lambda i: (i, 0)),
        out_shape=jax.ShapeDtypeStruct((E, F_OUT), jnp.float32),
        compiler_params=_ARB,
    )(z3, s3, q3, g3r, be3r, wft, bfr, x_src)


def _epilogue(p2, d2, bias8):
    return pl.pallas_call(
        _epi_body,
        out_shape=jax.ShapeDtypeStruct((N // 8, 128), jnp.float32),
    )(p2, d2, bias8)


@functools.lru_cache(maxsize=None)
def _sc_kernels():
    mesh = plsc.VectorSubcoreMesh(core_axis_name="c", subcore_axis_name="s")

    @functools.partial(
        pl.kernel,
        mesh=mesh,
        out_type=jax.ShapeDtypeStruct((E, F_IN), jnp.float32),
        scratch_types=[pltpu.VMEM((SCCH,), jnp.int32),
                       pltpu.VMEM((SCCH, F_IN), jnp.float32),
                       pltpu.SemaphoreType.DMA],
        compiler_params=pltpu.CompilerParams(use_tc_tiling_on_sc=False),
    )
    def gather(nfeat_hbm, src_hbm, out_hbm, idx_v, rows_v, sem):
        wid = lax.axis_index("s") * NC + lax.axis_index("c")
        base = wid * PER_W

        def body(j, carry):
            off = base + j * SCCH
            pltpu.sync_copy(src_hbm.at[pl.ds(off, SCCH)], idx_v)
            pltpu.async_copy(nfeat_hbm.at[idx_v], rows_v, sem).wait()
            pltpu.sync_copy(rows_v, out_hbm.at[pl.ds(off, SCCH)])
            return carry

        lax.fori_loop(0, NCHUNK, body, 0)

    @functools.partial(
        pl.kernel,
        mesh=mesh,
        out_type=[jax.ShapeDtypeStruct((NC, N, F_OUT), jnp.float32),
                  jax.ShapeDtypeStruct((NC, N, F_OUT), jnp.float32)],
        scratch_types=[pltpu.VMEM((SCCH,), jnp.int32),
                       pltpu.VMEM((SCCH, F_OUT), jnp.float32),
                       pltpu.VMEM((SCCH, F_OUT), jnp.float32),
                       pltpu.VMEM_SHARED((N, F_OUT), jnp.float32),
                       pltpu.VMEM_SHARED((N, F_OUT), jnp.float32)],
        compiler_params=pltpu.CompilerParams(use_tc_tiling_on_sc=False),
    )
    def scatter(msg_hbm, dst_hbm, zeros_hbm, ones_hbm,
                neigh_out, deg_out, idx_v, m_v, ones_v, acc_sh, deg_sh):
        c = lax.axis_index("c")
        s = lax.axis_index("s")
        wid = s * NC + c
        base = wid * PER_W

        @pl.when(s == 0)
        def _():
            pltpu.sync_copy(zeros_hbm, acc_sh)
            pltpu.sync_copy(zeros_hbm, deg_sh)

        pltpu.sync_copy(ones_hbm, ones_v)
        plsc.subcore_barrier()

        def body(j, carry):
            off = base + j * SCCH
            pltpu.sync_copy(dst_hbm.at[pl.ds(off, SCCH)], idx_v)
            pltpu.sync_copy(msg_hbm.at[pl.ds(off, SCCH)], m_v)
            pltpu.sync_copy(m_v, acc_sh.at[idx_v], add=True)
            pltpu.sync_copy(ones_v, deg_sh.at[idx_v], add=True)
            return carry

        lax.fori_loop(0, NCHUNK, body, 0)
        plsc.subcore_barrier()

        @pl.when(s == 0)
        def _():
            pltpu.sync_copy(acc_sh, neigh_out.at[c])
            pltpu.sync_copy(deg_sh, deg_out.at[c])

    return gather, scatter


def _sc_gather(nfeat, src):
    return _sc_kernels()[0](nfeat, src)


def _sc_scatter(msg, dst, zeros, ones):
    return _sc_kernels()[1](msg, dst, zeros, ones)


def kernel(nfeat, edge_index, efeat,
           W0, b0, g0, be0,
           W1, b1, g1, be1,
           W2, b2, g2, be2,
           W3, b3, g3, be3,
           Wf, bf, bias):
    src = edge_index[0]
    dst = edge_index[1]

    w0t = W0.T
    w1t = W1.T
    w2t = W2.T
    w3t = W3.T
    wft = Wf.T

    s0, q0 = _stats0(efeat, w0t, _row(b0))
    x_src = _sc_gather(nfeat, src)

    z1, s1, q1 = _layer1(efeat, w0t, _row(b0), s0, q0, _row(g0), _row(be0),
                         w1t, _row(b1))
    z2, s2, q2 = _mid(z1, s1, q1, _row(g1), _row(be1), w2t, _row(b2), 128, 64)
    z3, s3, q3 = _mid(z2, s2, q2, _row(g2), _row(be2), w3t, _row(b3), 64, 32)
    msg = _final(z3, s3, q3, _row(g3), _row(be3), wft, _row(bf), x_src)

    zeros = jnp.zeros((N, F_OUT), jnp.float32)
    ones = jnp.ones((SCCH, F_OUT), jnp.float32)
    part, degp = _sc_scatter(msg, dst, zeros, ones)

    p2 = part.reshape(NC, N // 8, 128)
    d2 = degp.reshape(NC, N // 8, 128)
    bias8 = jnp.tile(bias, 8).reshape(1, 128)
    out = _epilogue(p2, d2, bias8)
    return out.reshape(N, F_OUT)


# speedup vs baseline: 2.7371x; 1.1607x over previous
import functools

import jax
import jax.numpy as jnp
from jax import lax
from jax.experimental import pallas as pl
from jax.experimental.pallas import tpu as pltpu
from jax.experimental.pallas import tpu_sc as plsc

N = 10000
E = 320000
F_IN = 16
F_OUT = 16
EPS = 1e-5

TE = 4000
GRID = E // TE

NC = 2
NS = 16
NW = NC * NS
PER_W = E // NW
SCCH = 2000
NCHUNK = PER_W // SCCH

_ARB = pltpu.CompilerParams(dimension_semantics=("arbitrary",))


def _z0(ef, w0t_ref, b0_ref):
    return (ef[:, 0:1] * w0t_ref[0:1, :]
            + ef[:, 1:2] * w0t_ref[1:2, :]
            + ef[:, 2:3] * w0t_ref[2:3, :]
            + b0_ref[...])


def _acc_stats(i, z, s_ref, q_ref):
    ps = jnp.sum(z, axis=0, keepdims=True)
    pq = jnp.sum(z * z, axis=0, keepdims=True)

    @pl.when(i == 0)
    def _():
        s_ref[...] = ps
        q_ref[...] = pq

    @pl.when(i > 0)
    def _():
        s_ref[...] = s_ref[...] + ps
        q_ref[...] = q_ref[...] + pq


def _bn_relu(z, s_ref, q_ref, g_ref, be_ref):
    m = s_ref[...] * (1.0 / E)
    v = q_ref[...] * (1.0 / E) - m * m
    a = g_ref[...] * lax.rsqrt(v + EPS)
    c = be_ref[...] - m * a
    return jnp.maximum(z * a + c, 0.0)


def _stats0_body(ef_ref, w0t_ref, b0_ref, s_ref, q_ref):
    i = pl.program_id(0)
    z = _z0(ef_ref[...], w0t_ref, b0_ref)
    _acc_stats(i, z, s_ref, q_ref)


def _layer1_body(ef_ref, w0t_ref, b0_ref, s0_ref, q0_ref, g0_ref, be0_ref,
                 w1t_ref, b1_ref, z1_ref, s1_ref, q1_ref):
    i = pl.program_id(0)
    z0 = _z0(ef_ref[...], w0t_ref, b0_ref)
    h0 = _bn_relu(z0, s0_ref, q0_ref, g0_ref, be0_ref)
    z1 = jnp.dot(h0, w1t_ref[...], preferred_element_type=jnp.float32) + b1_ref[...]
    z1_ref[...] = z1
    _acc_stats(i, z1, s1_ref, q1_ref)


def _mid_body(z_ref, s_ref, q_ref, g_ref, be_ref, wt_ref, b_ref,
              zo_ref, so_ref, qo_ref):
    i = pl.program_id(0)
    h = _bn_relu(z_ref[...], s_ref, q_ref, g_ref, be_ref)
    z = jnp.dot(h, wt_ref[...], preferred_element_type=jnp.float32) + b_ref[...]
    zo_ref[...] = z
    _acc_stats(i, z, so_ref, qo_ref)


def _final_body(z3_ref, s_ref, q_ref, g_ref, be_ref, wft_ref, bf_ref,
                x_ref, msg_ref):
    h3 = _bn_relu(z3_ref[...], s_ref, q_ref, g_ref, be_ref)
    wv = jax.nn.sigmoid(
        jnp.dot(h3, wft_ref[...], preferred_element_type=jnp.float32)
        + bf_ref[...])
    x = x_ref[...]
    li = lax.broadcasted_iota(jnp.int32, (F_IN, F_IN * F_OUT), 1)
    ri = lax.broadcasted_iota(jnp.int32, (F_IN, F_IN * F_OUT), 0)
    rmat = (li // F_OUT == ri).astype(jnp.float32)
    si = lax.broadcasted_iota(jnp.int32, (F_IN * F_OUT, F_OUT), 0)
    oi = lax.broadcasted_iota(jnp.int32, (F_IN * F_OUT, F_OUT), 1)
    smat = (si % F_OUT == oi).astype(jnp.float32)
    xrep = jnp.dot(x, rmat, preferred_element_type=jnp.float32)
    msg_ref[...] = jnp.dot(xrep * wv, smat,
                           preferred_element_type=jnp.float32)


def _epi_body(p_ref, d_ref, bias_ref, out_ref):
    p = p_ref[...]
    d = d_ref[...]
    deg = jnp.maximum(d[0] + d[1], 1.0)
    out_ref[...] = (p[0] + p[1]) / deg + bias_ref[...]


def _row(x):
    return x.reshape(1, -1)


def _const_spec(shape):
    return pl.BlockSpec(shape, lambda i: (0,) * len(shape))


def _stats_out(ch):
    return ([jax.ShapeDtypeStruct((1, ch), jnp.float32)] * 2,
            [_const_spec((1, ch))] * 2)


def _stats0(efeat, w0t, b0r):
    out_shape, out_specs = _stats_out(256)
    return pl.pallas_call(
        _stats0_body,
        grid=(GRID,),
        in_specs=[pl.BlockSpec((TE, 3), lambda i: (i, 0)),
                  _const_spec((3, 256)), _const_spec((1, 256))],
        out_specs=out_specs,
        out_shape=out_shape,
        compiler_params=_ARB,
    )(efeat, w0t, b0r)


def _layer1(efeat, w0t, b0r, s0, q0, g0r, be0r, w1t, b1r):
    out_shape = [jax.ShapeDtypeStruct((E, 128), jnp.float32)]
    out_specs = [pl.BlockSpec((TE, 128), lambda i: (i, 0))]
    so, sspec = _stats_out(128)
    return pl.pallas_call(
        _layer1_body,
        grid=(GRID,),
        in_specs=[pl.BlockSpec((TE, 3), lambda i: (i, 0)),
                  _const_spec((3, 256)), _const_spec((1, 256)),
                  _const_spec((1, 256)), _const_spec((1, 256)),
                  _const_spec((1, 256)), _const_spec((1, 256)),
                  _const_spec((256, 128)), _const_spec((1, 128))],
        out_specs=out_specs + sspec,
        out_shape=out_shape + so,
        compiler_params=_ARB,
    )(efeat, w0t, b0r, s0, q0, g0r, be0r, w1t, b1r)


def _mid(z, s, q, gr, ber, wt, br, cin, cout):
    out_shape = [jax.ShapeDtypeStruct((E, cout), jnp.float32)]
    out_specs = [pl.BlockSpec((TE, cout), lambda i: (i, 0))]
    so, sspec = _stats_out(cout)
    return pl.pallas_call(
        _mid_body,
        grid=(GRID,),
        in_specs=[pl.BlockSpec((TE, cin), lambda i: (i, 0)),
                  _const_spec((1, cin)), _const_spec((1, cin)),
                  _const_spec((1, cin)), _const_spec((1, cin)),
                  _const_spec((cin, cout)), _const_spec((1, cout))],
        out_specs=out_specs + sspec,
        out_shape=out_shape + so,
        compiler_params=_ARB,
    )(z, s, q, gr, ber, wt, br)


def _final(z3, s3, q3, g3r, be3r, wft, bfr, x_src):
    return pl.pallas_call(
        _final_body,
        grid=(GRID,),
        in_specs=[pl.BlockSpec((TE, 32), lambda i: (i, 0)),
                  _const_spec((1, 32)), _const_spec((1, 32)),
                  _const_spec((1, 32)), _const_spec((1, 32)),
                  _const_spec((32, 256)), _const_spec((1, 256)),
                  pl.BlockSpec((TE, F_IN), lambda i: (i, 0))],
        out_specs=pl.BlockSpec((TE, F_OUT), lambda i: (i, 0)),
        out_shape=jax.ShapeDtypeStruct((E, F_OUT), jnp.float32),
        compiler_params=_ARB,
    )(z3, s3, q3, g3r, be3r, wft, bfr, x_src)


def _epilogue(p2, d2, bias8):
    return pl.pallas_call(
        _epi_body,
        out_shape=jax.ShapeDtypeStruct((N // 8, 128), jnp.float32),
    )(p2, d2, bias8)


@functools.lru_cache(maxsize=None)
def _sc_kernels():
    mesh = plsc.VectorSubcoreMesh(core_axis_name="c", subcore_axis_name="s")

    @functools.partial(
        pl.kernel,
        mesh=mesh,
        out_type=jax.ShapeDtypeStruct((E, F_IN), jnp.float32),
        scratch_types=[pltpu.VMEM((SCCH,), jnp.int32),
                       pltpu.VMEM((SCCH, F_IN), jnp.float32),
                       pltpu.SemaphoreType.DMA],
        compiler_params=pltpu.CompilerParams(use_tc_tiling_on_sc=False),
    )
    def gather(nfeat_hbm, src_hbm, out_hbm, idx_v, rows_v, sem):
        wid = lax.axis_index("s") * NC + lax.axis_index("c")
        base = wid * PER_W

        def body(j, carry):
            off = base + j * SCCH
            pltpu.sync_copy(src_hbm.at[pl.ds(off, SCCH)], idx_v)
            pltpu.async_copy(nfeat_hbm.at[idx_v], rows_v, sem).wait()
            pltpu.sync_copy(rows_v, out_hbm.at[pl.ds(off, SCCH)])
            return carry

        lax.fori_loop(0, NCHUNK, body, 0)

    @functools.partial(
        pl.kernel,
        mesh=mesh,
        out_type=[jax.ShapeDtypeStruct((NC, N, F_OUT), jnp.float32),
                  jax.ShapeDtypeStruct((NC, N, F_OUT), jnp.float32)],
        scratch_types=[pltpu.VMEM((SCCH,), jnp.int32),
                       pltpu.VMEM((SCCH, F_OUT), jnp.float32),
                       pltpu.VMEM((SCCH, F_OUT), jnp.float32),
                       pltpu.VMEM_SHARED((N, F_OUT), jnp.float32),
                       pltpu.VMEM_SHARED((N, F_OUT), jnp.float32)],
        compiler_params=pltpu.CompilerParams(use_tc_tiling_on_sc=False),
    )
    def scatter(msg_hbm, dst_hbm, zeros_hbm, ones_hbm,
                neigh_out, deg_out, idx_v, m_v, ones_v, acc_sh, deg_sh):
        c = lax.axis_index("c")
        s = lax.axis_index("s")
        wid = s * NC + c
        base = wid * PER_W

        @pl.when(s == 0)
        def _():
            pltpu.sync_copy(zeros_hbm, acc_sh)
            pltpu.sync_copy(zeros_hbm, deg_sh)

        pltpu.sync_copy(ones_hbm, ones_v)
        plsc.subcore_barrier()

        def body(j, carry):
            off = base + j * SCCH
            pltpu.sync_copy(dst_hbm.at[pl.ds(off, SCCH)], idx_v)
            pltpu.sync_copy(msg_hbm.at[pl.ds(off, SCCH)], m_v)
            pltpu.sync_copy(m_v, acc_sh.at[idx_v], add=True)
            pltpu.sync_copy(ones_v, deg_sh.at[idx_v], add=True)
            return carry

        lax.fori_loop(0, NCHUNK, body, 0)
        plsc.subcore_barrier()

        @pl.when(s == 0)
        def _():
            pltpu.sync_copy(acc_sh, neigh_out.at[c])
            pltpu.sync_copy(deg_sh, deg_out.at[c])

    return gather, scatter


def _sc_gather(nfeat, src):
    return _sc_kernels()[0](nfeat, src)


def _sc_scatter(msg, dst, zeros, ones):
    return _sc_kernels()[1](msg, dst, zeros, ones)


def kernel(nfeat, edge_index, efeat,
           W0, b0, g0, be0,
           W1, b1, g1, be1,
           W2, b2, g2, be2,
           W3, b3, g3, be3,
           Wf, bf, bias):
    src = edge_index[0]
    dst = edge_index[1]

    w0t = W0.T
    w1t = W1.T
    w2t = W2.T
    w3t = W3.T
    wft = Wf.T

    s0, q0 = _stats0(efeat, w0t, _row(b0))
    x_src = _sc_gather(nfeat, src)

    z1, s1, q1 = _layer1(efeat, w0t, _row(b0), s0, q0, _row(g0), _row(be0),
                         w1t, _row(b1))
    z2, s2, q2 = _mid(z1, s1, q1, _row(g1), _row(be1), w2t, _row(b2), 128, 64)
    z3, s3, q3 = _mid(z2, s2, q2, _row(g2), _row(be2), w3t, _row(b3), 64, 32)
    msg = _final(z3, s3, q3, _row(g3), _row(be3), wft, _row(bf), x_src)

    zeros = jnp.zeros((N, F_OUT), jnp.float32)
    ones = jnp.ones((SCCH, F_OUT), jnp.float32)
    part, degp = _sc_scatter(msg, dst, zeros, ones)

    p2 = part.reshape(NC, N // 8, 128)
    d2 = degp.reshape(NC, N // 8, 128)
    bias8 = jnp.tile(bias, 8).reshape(1, 128)
    out = _epilogue(p2, d2, bias8)
    return out.reshape(N, F_OUT)


# speedup vs baseline: 2.9049x; 1.0613x over previous
import functools

import jax
import jax.numpy as jnp
from jax import lax
from jax.experimental import pallas as pl
from jax.experimental.pallas import tpu as pltpu
from jax.experimental.pallas import tpu_sc as plsc

N = 10000
E = 320000
F_IN = 16
F_OUT = 16
EPS = 1e-5

TE = 8000
GRID = E // TE

NC = 2
NS = 16
NW = NC * NS
PER_W = E // NW
SCCH = 2000
NCHUNK = PER_W // SCCH

_ARB = pltpu.CompilerParams(dimension_semantics=("arbitrary",))


def _z0(ef, w0t_ref, b0_ref):
    return (ef[:, 0:1] * w0t_ref[0:1, :]
            + ef[:, 1:2] * w0t_ref[1:2, :]
            + ef[:, 2:3] * w0t_ref[2:3, :]
            + b0_ref[...])


def _acc_stats(i, z, s_ref, q_ref):
    ps = jnp.sum(z, axis=0, keepdims=True)
    pq = jnp.sum(z * z, axis=0, keepdims=True)

    @pl.when(i == 0)
    def _():
        s_ref[...] = ps
        q_ref[...] = pq

    @pl.when(i > 0)
    def _():
        s_ref[...] = s_ref[...] + ps
        q_ref[...] = q_ref[...] + pq


def _bn_relu(z, s_ref, q_ref, g_ref, be_ref):
    m = s_ref[...] * (1.0 / E)
    v = q_ref[...] * (1.0 / E) - m * m
    a = g_ref[...] * lax.rsqrt(v + EPS)
    c = be_ref[...] - m * a
    return jnp.maximum(z * a + c, 0.0)


def _stats0_body(ef_ref, w0t_ref, b0_ref, s_ref, q_ref):
    i = pl.program_id(0)
    z = _z0(ef_ref[...], w0t_ref, b0_ref)
    _acc_stats(i, z, s_ref, q_ref)


def _layer1_body(ef_ref, w0t_ref, b0_ref, s0_ref, q0_ref, g0_ref, be0_ref,
                 w1t_ref, b1_ref, z1_ref, s1_ref, q1_ref):
    i = pl.program_id(0)
    z0 = _z0(ef_ref[...], w0t_ref, b0_ref)
    h0 = _bn_relu(z0, s0_ref, q0_ref, g0_ref, be0_ref)
    z1 = jnp.dot(h0, w1t_ref[...], preferred_element_type=jnp.float32) + b1_ref[...]
    z1_ref[...] = z1
    _acc_stats(i, z1, s1_ref, q1_ref)


def _mid_body(z_ref, s_ref, q_ref, g_ref, be_ref, wt_ref, b_ref,
              zo_ref, so_ref, qo_ref):
    i = pl.program_id(0)
    h = _bn_relu(z_ref[...], s_ref, q_ref, g_ref, be_ref)
    z = jnp.dot(h, wt_ref[...], preferred_element_type=jnp.float32) + b_ref[...]
    zo_ref[...] = z
    _acc_stats(i, z, so_ref, qo_ref)


def _final_body(z3_ref, s_ref, q_ref, g_ref, be_ref, wft_ref, bf_ref,
                x_ref, msg_ref):
    h3 = _bn_relu(z3_ref[...], s_ref, q_ref, g_ref, be_ref)
    wv = jax.nn.sigmoid(
        jnp.dot(h3, wft_ref[...], preferred_element_type=jnp.float32)
        + bf_ref[...])
    x = x_ref[...]
    li = lax.broadcasted_iota(jnp.int32, (F_IN, F_IN * F_OUT), 1)
    ri = lax.broadcasted_iota(jnp.int32, (F_IN, F_IN * F_OUT), 0)
    rmat = (li // F_OUT == ri).astype(jnp.float32)
    si = lax.broadcasted_iota(jnp.int32, (F_IN * F_OUT, F_OUT), 0)
    oi = lax.broadcasted_iota(jnp.int32, (F_IN * F_OUT, F_OUT), 1)
    smat = (si % F_OUT == oi).astype(jnp.float32)
    xrep = jnp.dot(x, rmat, preferred_element_type=jnp.float32)
    msg_ref[...] = jnp.dot(xrep * wv, smat,
                           preferred_element_type=jnp.float32)


def _epi_body(p_ref, d_ref, bias_ref, out_ref):
    p = p_ref[...]
    d = d_ref[...]
    deg = jnp.maximum(d[0] + d[1], 1.0)
    out_ref[...] = (p[0] + p[1]) / deg + bias_ref[...]


def _row(x):
    return x.reshape(1, -1)


def _const_spec(shape):
    return pl.BlockSpec(shape, lambda i: (0,) * len(shape))


def _stats_out(ch):
    return ([jax.ShapeDtypeStruct((1, ch), jnp.float32)] * 2,
            [_const_spec((1, ch))] * 2)


def _stats0(efeat, w0t, b0r):
    out_shape, out_specs = _stats_out(256)
    return pl.pallas_call(
        _stats0_body,
        grid=(GRID,),
        in_specs=[pl.BlockSpec((TE, 3), lambda i: (i, 0)),
                  _const_spec((3, 256)), _const_spec((1, 256))],
        out_specs=out_specs,
        out_shape=out_shape,
        compiler_params=_ARB,
    )(efeat, w0t, b0r)


def _layer1(efeat, w0t, b0r, s0, q0, g0r, be0r, w1t, b1r):
    out_shape = [jax.ShapeDtypeStruct((E, 128), jnp.float32)]
    out_specs = [pl.BlockSpec((TE, 128), lambda i: (i, 0))]
    so, sspec = _stats_out(128)
    return pl.pallas_call(
        _layer1_body,
        grid=(GRID,),
        in_specs=[pl.BlockSpec((TE, 3), lambda i: (i, 0)),
                  _const_spec((3, 256)), _const_spec((1, 256)),
                  _const_spec((1, 256)), _const_spec((1, 256)),
                  _const_spec((1, 256)), _const_spec((1, 256)),
                  _const_spec((256, 128)), _const_spec((1, 128))],
        out_specs=out_specs + sspec,
        out_shape=out_shape + so,
        compiler_params=_ARB,
    )(efeat, w0t, b0r, s0, q0, g0r, be0r, w1t, b1r)


def _mid(z, s, q, gr, ber, wt, br, cin, cout):
    out_shape = [jax.ShapeDtypeStruct((E, cout), jnp.float32)]
    out_specs = [pl.BlockSpec((TE, cout), lambda i: (i, 0))]
    so, sspec = _stats_out(cout)
    return pl.pallas_call(
        _mid_body,
        grid=(GRID,),
        in_specs=[pl.BlockSpec((TE, cin), lambda i: (i, 0)),
                  _const_spec((1, cin)), _const_spec((1, cin)),
                  _const_spec((1, cin)), _const_spec((1, cin)),
                  _const_spec((cin, cout)), _const_spec((1, cout))],
        out_specs=out_specs + sspec,
        out_shape=out_shape + so,
        compiler_params=_ARB,
    )(z, s, q, gr, ber, wt, br)


def _final(z3, s3, q3, g3r, be3r, wft, bfr, x_src):
    return pl.pallas_call(
        _final_body,
        grid=(GRID,),
        in_specs=[pl.BlockSpec((TE, 32), lambda i: (i, 0)),
                  _const_spec((1, 32)), _const_spec((1, 32)),
                  _const_spec((1, 32)), _const_spec((1, 32)),
                  _const_spec((32, 256)), _const_spec((1, 256)),
                  pl.BlockSpec((TE, F_IN), lambda i: (i, 0))],
        out_specs=pl.BlockSpec((TE, F_OUT), lambda i: (i, 0)),
        out_shape=jax.ShapeDtypeStruct((E, F_OUT), jnp.float32),
        compiler_params=_ARB,
    )(z3, s3, q3, g3r, be3r, wft, bfr, x_src)


def _epilogue(p2, d2, bias8):
    return pl.pallas_call(
        _epi_body,
        out_shape=jax.ShapeDtypeStruct((N // 8, 128), jnp.float32),
    )(p2, d2, bias8)


@functools.lru_cache(maxsize=None)
def _sc_kernels():
    mesh = plsc.VectorSubcoreMesh(core_axis_name="c", subcore_axis_name="s")

    @functools.partial(
        pl.kernel,
        mesh=mesh,
        out_type=jax.ShapeDtypeStruct((E, F_IN), jnp.float32),
        scratch_types=[pltpu.VMEM((SCCH,), jnp.int32),
                       pltpu.VMEM((SCCH, F_IN), jnp.float32),
                       pltpu.SemaphoreType.DMA],
        compiler_params=pltpu.CompilerParams(use_tc_tiling_on_sc=False),
    )
    def gather(nfeat_hbm, src_hbm, out_hbm, idx_v, rows_v, sem):
        wid = lax.axis_index("s") * NC + lax.axis_index("c")
        base = wid * PER_W

        def body(j, carry):
            off = base + j * SCCH
            pltpu.sync_copy(src_hbm.at[pl.ds(off, SCCH)], idx_v)
            pltpu.async_copy(nfeat_hbm.at[idx_v], rows_v, sem).wait()
            pltpu.sync_copy(rows_v, out_hbm.at[pl.ds(off, SCCH)])
            return carry

        lax.fori_loop(0, NCHUNK, body, 0)

    @functools.partial(
        pl.kernel,
        mesh=mesh,
        out_type=[jax.ShapeDtypeStruct((NC, N, F_OUT), jnp.float32),
                  jax.ShapeDtypeStruct((NC, N, F_OUT), jnp.float32)],
        scratch_types=[pltpu.VMEM((SCCH,), jnp.int32),
                       pltpu.VMEM((SCCH, F_OUT), jnp.float32),
                       pltpu.VMEM((SCCH, F_OUT), jnp.float32),
                       pltpu.VMEM_SHARED((N, F_OUT), jnp.float32),
                       pltpu.VMEM_SHARED((N, F_OUT), jnp.float32)],
        compiler_params=pltpu.CompilerParams(use_tc_tiling_on_sc=False),
    )
    def scatter(msg_hbm, dst_hbm, zeros_hbm, ones_hbm,
                neigh_out, deg_out, idx_v, m_v, ones_v, acc_sh, deg_sh):
        c = lax.axis_index("c")
        s = lax.axis_index("s")
        wid = s * NC + c
        base = wid * PER_W

        @pl.when(s == 0)
        def _():
            pltpu.sync_copy(zeros_hbm, acc_sh)
            pltpu.sync_copy(zeros_hbm, deg_sh)

        pltpu.sync_copy(ones_hbm, ones_v)
        plsc.subcore_barrier()

        def body(j, carry):
            off = base + j * SCCH
            pltpu.sync_copy(dst_hbm.at[pl.ds(off, SCCH)], idx_v)
            pltpu.sync_copy(msg_hbm.at[pl.ds(off, SCCH)], m_v)
            pltpu.sync_copy(m_v, acc_sh.at[idx_v], add=True)
            pltpu.sync_copy(ones_v, deg_sh.at[idx_v], add=True)
            return carry

        lax.fori_loop(0, NCHUNK, body, 0)
        plsc.subcore_barrier()

        @pl.when(s == 0)
        def _():
            pltpu.sync_copy(acc_sh, neigh_out.at[c])
            pltpu.sync_copy(deg_sh, deg_out.at[c])

    return gather, scatter


def _sc_gather(nfeat, src):
    return _sc_kernels()[0](nfeat, src)


def _sc_scatter(msg, dst, zeros, ones):
    return _sc_kernels()[1](msg, dst, zeros, ones)


def kernel(nfeat, edge_index, efeat,
           W0, b0, g0, be0,
           W1, b1, g1, be1,
           W2, b2, g2, be2,
           W3, b3, g3, be3,
           Wf, bf, bias):
    src = edge_index[0]
    dst = edge_index[1]

    w0t = W0.T
    w1t = W1.T
    w2t = W2.T
    w3t = W3.T
    wft = Wf.T

    s0, q0 = _stats0(efeat, w0t, _row(b0))
    x_src = _sc_gather(nfeat, src)

    z1, s1, q1 = _layer1(efeat, w0t, _row(b0), s0, q0, _row(g0), _row(be0),
                         w1t, _row(b1))
    z2, s2, q2 = _mid(z1, s1, q1, _row(g1), _row(be1), w2t, _row(b2), 128, 64)
    z3, s3, q3 = _mid(z2, s2, q2, _row(g2), _row(be2), w3t, _row(b3), 64, 32)
    msg = _final(z3, s3, q3, _row(g3), _row(be3), wft, _row(bf), x_src)

    zeros = jnp.zeros((N, F_OUT), jnp.float32)
    ones = jnp.ones((SCCH, F_OUT), jnp.float32)
    part, degp = _sc_scatter(msg, dst, zeros, ones)

    p2 = part.reshape(NC, N // 8, 128)
    d2 = degp.reshape(NC, N // 8, 128)
    bias8 = jnp.tile(bias, 8).reshape(1, 128)
    out = _epilogue(p2, d2, bias8)
    return out.reshape(N, F_OUT)


# speedup vs baseline: 2.9887x; 1.0289x over previous
import functools

import jax
import jax.numpy as jnp
from jax import lax
from jax.experimental import pallas as pl
from jax.experimental.pallas import tpu as pltpu
from jax.experimental.pallas import tpu_sc as plsc

N = 10000
E = 320000
F_IN = 16
F_OUT = 16
EPS = 1e-5

TE = 8000
GRID = E // TE

NC = 2
NS = 16
NW = NC * NS
PER_W = E // NW
SCCH = 2000
NCHUNK = PER_W // SCCH

_ARB = pltpu.CompilerParams(dimension_semantics=("arbitrary",))


def _z0(ef, w0t_ref, b0_ref):
    return (ef[:, 0:1] * w0t_ref[0:1, :]
            + ef[:, 1:2] * w0t_ref[1:2, :]
            + ef[:, 2:3] * w0t_ref[2:3, :]
            + b0_ref[...])


def _acc_stats(i, z, s_ref, q_ref):
    ps = jnp.sum(z, axis=0, keepdims=True)
    pq = jnp.sum(z * z, axis=0, keepdims=True)

    @pl.when(i == 0)
    def _():
        s_ref[...] = ps
        q_ref[...] = pq

    @pl.when(i > 0)
    def _():
        s_ref[...] = s_ref[...] + ps
        q_ref[...] = q_ref[...] + pq


def _bn_relu(z, s_ref, q_ref, g_ref, be_ref):
    m = s_ref[...] * (1.0 / E)
    v = q_ref[...] * (1.0 / E) - m * m
    a = g_ref[...] * lax.rsqrt(v + EPS)
    c = be_ref[...] - m * a
    return jnp.maximum(z * a + c, 0.0)


def _moments_body(ef_ref, ms_ref, m2_ref):
    i = pl.program_id(0)
    e = ef_ref[...]
    ps = jnp.sum(e, axis=0, keepdims=True)
    rows = [jnp.sum(e * e[:, k:k + 1], axis=0, keepdims=True) for k in range(3)]
    pm2 = jnp.concatenate(rows, axis=0)

    @pl.when(i == 0)
    def _():
        ms_ref[...] = ps
        m2_ref[...] = pm2

    @pl.when(i > 0)
    def _():
        ms_ref[...] = ms_ref[...] + ps
        m2_ref[...] = m2_ref[...] + pm2


def _layer1_body(ef_ref, w0t_ref, b0_ref, ms_ref, m2_ref, g0_ref, be0_ref,
                 w1t_ref, b1_ref, z1_ref, s1_ref, q1_ref):
    i = pl.program_id(0)
    z0 = _z0(ef_ref[...], w0t_ref, b0_ref)
    inv_e = 1.0 / E
    mu = [ms_ref[0, j] * inv_e for j in range(3)]
    wrow = [w0t_ref[j:j + 1, :] for j in range(3)]
    m0 = mu[0] * wrow[0] + mu[1] * wrow[1] + mu[2] * wrow[2] + b0_ref[...]
    var = jnp.zeros_like(m0)
    for j in range(3):
        for k in range(3):
            cjk = m2_ref[j, k] * inv_e - mu[j] * mu[k]
            var = var + cjk * (wrow[j] * wrow[k])
    a0 = g0_ref[...] * lax.rsqrt(var + EPS)
    c0 = be0_ref[...] - m0 * a0
    h0 = jnp.maximum(z0 * a0 + c0, 0.0)
    z1 = jnp.dot(h0, w1t_ref[...], preferred_element_type=jnp.float32) + b1_ref[...]
    z1_ref[...] = z1
    _acc_stats(i, z1, s1_ref, q1_ref)


def _mid_body(z_ref, s_ref, q_ref, g_ref, be_ref, wt_ref, b_ref,
              zo_ref, so_ref, qo_ref):
    i = pl.program_id(0)
    h = _bn_relu(z_ref[...], s_ref, q_ref, g_ref, be_ref)
    z = jnp.dot(h, wt_ref[...], preferred_element_type=jnp.float32) + b_ref[...]
    zo_ref[...] = z
    _acc_stats(i, z, so_ref, qo_ref)


def _final_body(z3_ref, s_ref, q_ref, g_ref, be_ref, wft_ref, bf_ref,
                x_ref, msg_ref):
    h3 = _bn_relu(z3_ref[...], s_ref, q_ref, g_ref, be_ref)
    wv = jax.nn.sigmoid(
        jnp.dot(h3, wft_ref[...], preferred_element_type=jnp.float32)
        + bf_ref[...])
    x = x_ref[...]
    li = lax.broadcasted_iota(jnp.int32, (F_IN, F_IN * F_OUT), 1)
    ri = lax.broadcasted_iota(jnp.int32, (F_IN, F_IN * F_OUT), 0)
    rmat = (li // F_OUT == ri).astype(jnp.float32)
    si = lax.broadcasted_iota(jnp.int32, (F_IN * F_OUT, F_OUT), 0)
    oi = lax.broadcasted_iota(jnp.int32, (F_IN * F_OUT, F_OUT), 1)
    smat = (si % F_OUT == oi).astype(jnp.float32)
    xrep = jnp.dot(x, rmat, preferred_element_type=jnp.float32)
    msg_ref[...] = jnp.dot(xrep * wv, smat,
                           preferred_element_type=jnp.float32)


def _epi_body(p_ref, d_ref, bias_ref, out_ref):
    p = p_ref[...]
    d = d_ref[...]
    deg = jnp.maximum(d[0] + d[1], 1.0)
    out_ref[...] = (p[0] + p[1]) / deg + bias_ref[...]


def _row(x):
    return x.reshape(1, -1)


def _const_spec(shape):
    return pl.BlockSpec(shape, lambda i: (0,) * len(shape))


def _stats_out(ch):
    return ([jax.ShapeDtypeStruct((1, ch), jnp.float32)] * 2,
            [_const_spec((1, ch))] * 2)


def _moments(efeat):
    return pl.pallas_call(
        _moments_body,
        grid=(GRID,),
        in_specs=[pl.BlockSpec((TE, 3), lambda i: (i, 0))],
        out_specs=[_const_spec((1, 3)), _const_spec((3, 3))],
        out_shape=[jax.ShapeDtypeStruct((1, 3), jnp.float32),
                   jax.ShapeDtypeStruct((3, 3), jnp.float32)],
        compiler_params=_ARB,
    )(efeat)


def _layer1(efeat, w0t, b0r, s0, q0, g0r, be0r, w1t, b1r):
    out_shape = [jax.ShapeDtypeStruct((E, 128), jnp.float32)]
    out_specs = [pl.BlockSpec((TE, 128), lambda i: (i, 0))]
    so, sspec = _stats_out(128)
    return pl.pallas_call(
        _layer1_body,
        grid=(GRID,),
        in_specs=[pl.BlockSpec((TE, 3), lambda i: (i, 0)),
                  _const_spec((3, 256)), _const_spec((1, 256)),
                  _const_spec((1, 3)), _const_spec((3, 3)),
                  _const_spec((1, 256)), _const_spec((1, 256)),
                  _const_spec((256, 128)), _const_spec((1, 128))],
        out_specs=out_specs + sspec,
        out_shape=out_shape + so,
        compiler_params=_ARB,
    )(efeat, w0t, b0r, s0, q0, g0r, be0r, w1t, b1r)


def _mid(z, s, q, gr, ber, wt, br, cin, cout):
    out_shape = [jax.ShapeDtypeStruct((E, cout), jnp.float32)]
    out_specs = [pl.BlockSpec((TE, cout), lambda i: (i, 0))]
    so, sspec = _stats_out(cout)
    return pl.pallas_call(
        _mid_body,
        grid=(GRID,),
        in_specs=[pl.BlockSpec((TE, cin), lambda i: (i, 0)),
                  _const_spec((1, cin)), _const_spec((1, cin)),
                  _const_spec((1, cin)), _const_spec((1, cin)),
                  _const_spec((cin, cout)), _const_spec((1, cout))],
        out_specs=out_specs + sspec,
        out_shape=out_shape + so,
        compiler_params=_ARB,
    )(z, s, q, gr, ber, wt, br)


def _final(z3, s3, q3, g3r, be3r, wft, bfr, x_src):
    return pl.pallas_call(
        _final_body,
        grid=(GRID,),
        in_specs=[pl.BlockSpec((TE, 32), lambda i: (i, 0)),
                  _const_spec((1, 32)), _const_spec((1, 32)),
                  _const_spec((1, 32)), _const_spec((1, 32)),
                  _const_spec((32, 256)), _const_spec((1, 256)),
                  pl.BlockSpec((TE, F_IN), lambda i: (i, 0))],
        out_specs=pl.BlockSpec((TE, F_OUT), lambda i: (i, 0)),
        out_shape=jax.ShapeDtypeStruct((E, F_OUT), jnp.float32),
        compiler_params=_ARB,
    )(z3, s3, q3, g3r, be3r, wft, bfr, x_src)


def _epilogue(p2, d2, bias8):
    return pl.pallas_call(
        _epi_body,
        out_shape=jax.ShapeDtypeStruct((N // 8, 128), jnp.float32),
    )(p2, d2, bias8)


@functools.lru_cache(maxsize=None)
def _sc_kernels():
    mesh = plsc.VectorSubcoreMesh(core_axis_name="c", subcore_axis_name="s")

    @functools.partial(
        pl.kernel,
        mesh=mesh,
        out_type=jax.ShapeDtypeStruct((E, F_IN), jnp.float32),
        scratch_types=[pltpu.VMEM((SCCH,), jnp.int32),
                       pltpu.VMEM((SCCH, F_IN), jnp.float32),
                       pltpu.SemaphoreType.DMA],
        compiler_params=pltpu.CompilerParams(use_tc_tiling_on_sc=False),
    )
    def gather(nfeat_hbm, src_hbm, out_hbm, idx_v, rows_v, sem):
        wid = lax.axis_index("s") * NC + lax.axis_index("c")
        base = wid * PER_W

        def body(j, carry):
            off = base + j * SCCH
            pltpu.sync_copy(src_hbm.at[pl.ds(off, SCCH)], idx_v)
            pltpu.async_copy(nfeat_hbm.at[idx_v], rows_v, sem).wait()
            pltpu.sync_copy(rows_v, out_hbm.at[pl.ds(off, SCCH)])
            return carry

        lax.fori_loop(0, NCHUNK, body, 0)

    @functools.partial(
        pl.kernel,
        mesh=mesh,
        out_type=[jax.ShapeDtypeStruct((NC, N, F_OUT), jnp.float32),
                  jax.ShapeDtypeStruct((NC, N, F_OUT), jnp.float32)],
        scratch_types=[pltpu.VMEM((SCCH,), jnp.int32),
                       pltpu.VMEM((SCCH, F_OUT), jnp.float32),
                       pltpu.VMEM((SCCH, F_OUT), jnp.float32),
                       pltpu.VMEM_SHARED((N, F_OUT), jnp.float32),
                       pltpu.VMEM_SHARED((N, F_OUT), jnp.float32)],
        compiler_params=pltpu.CompilerParams(use_tc_tiling_on_sc=False),
    )
    def scatter(msg_hbm, dst_hbm, zeros_hbm, ones_hbm,
                neigh_out, deg_out, idx_v, m_v, ones_v, acc_sh, deg_sh):
        c = lax.axis_index("c")
        s = lax.axis_index("s")
        wid = s * NC + c
        base = wid * PER_W

        @pl.when(s == 0)
        def _():
            pltpu.sync_copy(zeros_hbm, acc_sh)
            pltpu.sync_copy(zeros_hbm, deg_sh)

        pltpu.sync_copy(ones_hbm, ones_v)
        plsc.subcore_barrier()

        def body(j, carry):
            off = base + j * SCCH
            pltpu.sync_copy(dst_hbm.at[pl.ds(off, SCCH)], idx_v)
            pltpu.sync_copy(msg_hbm.at[pl.ds(off, SCCH)], m_v)
            pltpu.sync_copy(m_v, acc_sh.at[idx_v], add=True)
            pltpu.sync_copy(ones_v, deg_sh.at[idx_v], add=True)
            return carry

        lax.fori_loop(0, NCHUNK, body, 0)
        plsc.subcore_barrier()

        @pl.when(s == 0)
        def _():
            pltpu.sync_copy(acc_sh, neigh_out.at[c])
            pltpu.sync_copy(deg_sh, deg_out.at[c])

    return gather, scatter


def _sc_gather(nfeat, src):
    return _sc_kernels()[0](nfeat, src)


def _sc_scatter(msg, dst, zeros, ones):
    return _sc_kernels()[1](msg, dst, zeros, ones)


def kernel(nfeat, edge_index, efeat,
           W0, b0, g0, be0,
           W1, b1, g1, be1,
           W2, b2, g2, be2,
           W3, b3, g3, be3,
           Wf, bf, bias):
    src = edge_index[0]
    dst = edge_index[1]

    w0t = W0.T
    w1t = W1.T
    w2t = W2.T
    w3t = W3.T
    wft = Wf.T

    ms, m2 = _moments(efeat)
    x_src = _sc_gather(nfeat, src)

    z1, s1, q1 = _layer1(efeat, w0t, _row(b0), ms, m2, _row(g0), _row(be0),
                         w1t, _row(b1))
    z2, s2, q2 = _mid(z1, s1, q1, _row(g1), _row(be1), w2t, _row(b2), 128, 64)
    z3, s3, q3 = _mid(z2, s2, q2, _row(g2), _row(be2), w3t, _row(b3), 64, 32)
    msg = _final(z3, s3, q3, _row(g3), _row(be3), wft, _row(bf), x_src)

    zeros = jnp.zeros((N, F_OUT), jnp.float32)
    ones = jnp.ones((SCCH, F_OUT), jnp.float32)
    part, degp = _sc_scatter(msg, dst, zeros, ones)

    p2 = part.reshape(NC, N // 8, 128)
    d2 = degp.reshape(NC, N // 8, 128)
    bias8 = jnp.tile(bias, 8).reshape(1, 128)
    out = _epilogue(p2, d2, bias8)
    return out.reshape(N, F_OUT)
